# baseline (speedup 1.0000x reference)
import functools

import jax
import jax.numpy as jnp
from jax import lax
from jax.experimental import pallas as pl
from jax.experimental.pallas import tpu as pltpu
from jax.experimental.pallas import tpu_sc as plsc

_NBR = 5
_NCOMP = 32
_FWB = 32
_N = 1024
_NW = 32
_RPW = _N // _NW
_LANE = 16


_CAND = (_NBR + 1) * _LANE


@functools.partial(
    pl.kernel,
    mesh=plsc.VectorSubcoreMesh(core_axis_name="c", subcore_axis_name="s"),
    out_type=[
        jax.ShapeDtypeStruct((_N, _CAND), jnp.float32),
        jax.ShapeDtypeStruct((_N, _CAND), jnp.int32),
    ],
    scratch_types=[
        pltpu.VMEM((_RPW, _N), jnp.float32),
        pltpu.VMEM((_RPW, _CAND), jnp.float32),
        pltpu.VMEM((_RPW, _CAND), jnp.int32),
    ],
)
def _sc_knn(dist_hbm, cval_hbm, cidx_hbm, rows_v, cval_v, cidx_v):
    wid = lax.axis_index("s") * 2 + lax.axis_index("c")
    base = wid * _RPW
    lanes = lax.iota(jnp.int32, _LANE)
    bigf = jnp.full((_LANE,), 3.0e38, jnp.float32)
    zero = jnp.zeros((_LANE,), jnp.int32)
    pltpu.sync_copy(dist_hbm.at[pl.ds(base, _RPW)], rows_v)

    def row_body(r, carry):
        def chunk_body(c, mc):
            ms, idxs = mc
            v = rows_v[r, pl.ds(c * _LANE, _LANE)]
            gi = c * _LANE + lanes
            bs = [v < mk for mk in ms]
            nms, nis = [], []
            for k in range(_NBR, 0, -1):
                nms.insert(0, jnp.where(bs[k - 1], ms[k - 1],
                                        jnp.where(bs[k], v, ms[k])))
                nis.insert(0, jnp.where(bs[k - 1], idxs[k - 1],
                                        jnp.where(bs[k], gi, idxs[k])))
            nms.insert(0, jnp.where(bs[0], v, ms[0]))
            nis.insert(0, jnp.where(bs[0], gi, idxs[0]))
            return (nms, nis)

        ms, idxs = lax.fori_loop(
            0, _N // _LANE, chunk_body,
            ([bigf] * (_NBR + 1), [zero] * (_NBR + 1)))
        for t in range(_NBR + 1):
            cval_v[r, pl.ds(t * _LANE, _LANE)] = ms[t]
            cidx_v[r, pl.ds(t * _LANE, _LANE)] = idxs[t]
        return carry

    lax.fori_loop(0, _RPW, row_body, 0)
    pltpu.sync_copy(cval_v, cval_hbm.at[pl.ds(base, _RPW)])
    pltpu.sync_copy(cidx_v, cidx_hbm.at[pl.ds(base, _RPW)])


def _graph_body(dist_ref, cval_ref, cidx_ref, d_ref):
    n = dist_ref.shape[0]
    inf = jnp.float32(jnp.inf)
    dist = dist_ref[...]
    d_ref[...] = jnp.full((n, n), inf, jnp.float32)
    colidx = jax.lax.broadcasted_iota(jnp.int32, (n, n), 1)

    vals = cval_ref[...]
    idxs = cidx_ref[...]
    big = jnp.float32(3.0e38)
    for t in range(_NBR + 1):
        rowmin = jnp.min(vals, axis=1, keepdims=True)
        sel = vals == rowmin
        widx = jnp.min(jnp.where(sel, idxs, n), axis=1, keepdims=True)
        if t > 0:
            d_ref[...] = jnp.where(colidx == widx, dist, d_ref[...])
        vals = jnp.where(idxs == widx, big, vals)

    a = jnp.minimum(d_ref[...], jnp.transpose(d_ref[...]))
    eye = jax.lax.broadcasted_iota(jnp.int32, (n, n), 0) == colidx
    d_ref[...] = jnp.where(eye, 0.0, a)

    def fw_block(kb, carry):
        k0 = kb * _FWB
        p = d_ref[pl.ds(k0, _FWB), :]
        pd = pltpu.roll(p, -k0, 1)[:, :_FWB]
        snaps = []
        for t in range(_FWB):
            rowt = p[t:t + 1, :]
            snaps.append(rowt)
            colt = pd[:, t:t + 1]
            p = jnp.minimum(p, colt + rowt)
            pd = jnp.minimum(pd, colt + pd[t:t + 1, :])
        s = jnp.concatenate(snaps, axis=0)
        st = jnp.transpose(s)
        upd = st[:, 0:1] + s[0:1, :]
        for t in range(1, _FWB):
            upd = jnp.minimum(upd, st[:, t:t + 1] + s[t:t + 1, :])
        d_ref[...] = jnp.minimum(d_ref[...], upd)
        return carry

    jax.lax.fori_loop(0, n // _FWB, fw_block, 0)


def kernel(toLearn):
    flat = toLearn.reshape(toLearn.shape[0], -1)
    n = flat.shape[0]
    sq = jnp.sum(flat * flat, axis=1)
    d2 = sq[:, None] + sq[None, :] - 2.0 * (flat @ flat.T)
    d2 = jnp.maximum(d2, 0.0)
    dist = jnp.sqrt(d2)

    cval, cidx = _sc_knn(dist)
    D = pl.pallas_call(
        _graph_body,
        out_shape=jax.ShapeDtypeStruct((n, n), jnp.float32),
    )(dist, cval, cidx)

    finite = jnp.isfinite(D)
    dmax = jnp.max(jnp.where(finite, D, 0.0))
    D = jnp.where(finite, D, dmax)
    D2 = D * D
    G = -0.5 * (D2 - D2.mean(axis=0, keepdims=True)
                - D2.mean(axis=1, keepdims=True) + D2.mean())
    w, v = jnp.linalg.eigh(G)
    w = w[::-1][:_NCOMP]
    v = v[:, ::-1][:, :_NCOMP]
    emb = v * jnp.sqrt(jnp.maximum(w, 0.0))[None, :]
    return emb.astype(jnp.float32)
